# baseline (speedup 1.0000x reference)
import functools

import jax
import jax.numpy as jnp
from jax import lax
from jax.experimental import pallas as pl
from jax.experimental.pallas import tpu as pltpu
from jax.experimental.pallas import tpu_sc as plsc

VOCAB = 100000
D = 128
NQ = 64
ROW_BLK = 2000
OUT_BLK = 4096

NW = 32
CHUNK = 128
FIRE = 4


def _tanh_layer_kernel(emb_ref, w1_ref, b1_ref, h_ref):
    h_ref[...] = jnp.tanh(
        jnp.dot(emb_ref[...], w1_ref[...], preferred_element_type=jnp.float32,
                precision=lax.Precision.HIGHEST)
        + b1_ref[...]
    )


def _tanh_layer(emb, W1, b1):
    grid = VOCAB // ROW_BLK
    return pl.pallas_call(
        _tanh_layer_kernel,
        grid=(grid,),
        in_specs=[
            pl.BlockSpec((ROW_BLK, D), lambda i: (i, 0)),
            pl.BlockSpec((D, D), lambda i: (0, 0)),
            pl.BlockSpec((1, D), lambda i: (0, 0)),
        ],
        out_specs=pl.BlockSpec((ROW_BLK, D), lambda i: (i, 0)),
        out_shape=jax.ShapeDtypeStruct((VOCAB, D), jnp.float32),
    )(emb, W1, b1.reshape(1, D))


def _head_kernel(g_ref, w2_ref, b2_ref, o_ref):
    nseq = o_ref.shape[0]
    o_ref[...] = (
        jnp.dot(g_ref[...], w2_ref[...], preferred_element_type=jnp.float32,
                precision=lax.Precision.HIGHEST)
        + b2_ref[...]
    ).reshape(nseq, -1, NQ)


def _head(g, W2, b2, B, L):
    SEQ_BLK = 32
    grid = B // SEQ_BLK
    return pl.pallas_call(
        _head_kernel,
        grid=(grid,),
        in_specs=[
            pl.BlockSpec((SEQ_BLK * L, D), lambda i: (i, 0)),
            pl.BlockSpec((D, NQ), lambda i: (0, 0)),
            pl.BlockSpec((1, NQ), lambda i: (0, 0)),
        ],
        out_specs=pl.BlockSpec((SEQ_BLK, L, NQ), lambda i: (i, 0, 0)),
        out_shape=jax.ShapeDtypeStruct((B, L, NQ), jnp.float32),
    )(g, W2, b2.reshape(1, NQ))


def _make_sc_gather(n_tokens):
    per_w = n_tokens // NW
    n_steps = per_w // (FIRE * CHUNK)
    idx_rows = per_w // CHUNK

    mesh = plsc.VectorSubcoreMesh(core_axis_name="c", subcore_axis_name="s")
    info = plsc.get_sparse_core_info()
    nc = info.num_cores

    @functools.partial(
        pl.kernel,
        out_type=jax.ShapeDtypeStruct((n_tokens, D), jnp.float32),
        mesh=mesh,
        scratch_types=[
            pltpu.VMEM((idx_rows, CHUNK), jnp.int32),
            pltpu.VMEM((FIRE * CHUNK, D), jnp.float32),
            pltpu.SemaphoreType.DMA,
        ],
    )
    def gather_kernel(table_hbm, idx_hbm, out_hbm, idx_v, rows_v, sem):
        wid = lax.axis_index("s") * nc + lax.axis_index("c")
        base = wid * per_w
        pltpu.sync_copy(idx_hbm.at[pl.ds(wid * idx_rows, idx_rows)], idx_v)

        def step(g, carry):
            copies = []
            for b in range(FIRE):
                j = g * FIRE + b
                copies.append(
                    pltpu.async_copy(
                        table_hbm.at[idx_v.at[j]],
                        rows_v.at[pl.ds(b * CHUNK, CHUNK)],
                        sem,
                    )
                )
            for c in copies:
                c.wait()
            pltpu.sync_copy(
                rows_v,
                out_hbm.at[pl.ds(base + g * (FIRE * CHUNK), FIRE * CHUNK)],
            )
            return carry

        lax.fori_loop(0, n_steps, step, 0)

    return gather_kernel


def kernel(tok, emb, W1, b1, W2, b2):
    B, L = tok.shape
    n_tokens = B * L
    table = _tanh_layer(emb, W1, b1)
    idx2d = tok.reshape(n_tokens // CHUNK, CHUNK).astype(jnp.int32)
    g = _make_sc_gather(n_tokens)(table, idx2d)
    return _head(g, W2, b2, B, L)


# speedup vs baseline: 1.5547x; 1.5547x over previous
import functools

import jax
import jax.numpy as jnp
from jax import lax
from jax.experimental import pallas as pl
from jax.experimental.pallas import tpu as pltpu
from jax.experimental.pallas import tpu_sc as plsc

VOCAB = 100000
D = 128
NQ = 64
ROW_BLK = 2000
OUT_BLK = 4096

NW = 32
CHUNK = 128
FIRE = 4


def _tanh_layer_kernel(emb_ref, w1_ref, b1_ref, h_ref):
    h_ref[...] = jnp.tanh(
        jnp.dot(emb_ref[...], w1_ref[...], preferred_element_type=jnp.float32,
                precision=lax.Precision.DEFAULT)
        + b1_ref[...]
    )


def _tanh_layer(emb, W1, b1):
    grid = VOCAB // ROW_BLK
    return pl.pallas_call(
        _tanh_layer_kernel,
        grid=(grid,),
        in_specs=[
            pl.BlockSpec((ROW_BLK, D), lambda i: (i, 0)),
            pl.BlockSpec((D, D), lambda i: (0, 0)),
            pl.BlockSpec((1, D), lambda i: (0, 0)),
        ],
        out_specs=pl.BlockSpec((ROW_BLK, D), lambda i: (i, 0)),
        out_shape=jax.ShapeDtypeStruct((VOCAB, D), jnp.float32),
    )(emb, W1, b1.reshape(1, D))


def _head_kernel(g_ref, w2_ref, b2_ref, o_ref):
    l_blk = o_ref.shape[0]
    for l in range(l_blk):
        acc = lax.dot_general(
            w2_ref[...], g_ref[:, l, :],
            (((0,), (1,)), ((), ())),
            preferred_element_type=jnp.float32,
            precision=lax.Precision.DEFAULT,
        )
        o_ref[l] = acc + b2_ref[...]


def _head(g, W2, b2, B, L):
    L_BLK = 8
    B_BLK = 1024
    g3 = g.reshape(B, L, D)
    return pl.pallas_call(
        _head_kernel,
        grid=(L // L_BLK, B // B_BLK),
        in_specs=[
            pl.BlockSpec((B_BLK, L_BLK, D), lambda i, j: (j, i, 0)),
            pl.BlockSpec((D, NQ), lambda i, j: (0, 0)),
            pl.BlockSpec((NQ, 1), lambda i, j: (0, 0)),
        ],
        out_specs=pl.BlockSpec((L_BLK, NQ, B_BLK), lambda i, j: (i, 0, j)),
        out_shape=jax.ShapeDtypeStruct((L, NQ, B), jnp.float32),
    )(g3, W2, b2.reshape(NQ, 1))


def _make_sc_gather(n_tokens):
    per_w = n_tokens // NW
    n_steps = per_w // (FIRE * CHUNK)
    idx_rows = per_w // CHUNK

    mesh = plsc.VectorSubcoreMesh(core_axis_name="c", subcore_axis_name="s")
    info = plsc.get_sparse_core_info()
    nc = info.num_cores

    @functools.partial(
        pl.kernel,
        out_type=jax.ShapeDtypeStruct((n_tokens, D), jnp.float32),
        mesh=mesh,
        scratch_types=[
            pltpu.VMEM((idx_rows, CHUNK), jnp.int32),
            pltpu.VMEM((FIRE * CHUNK, D), jnp.float32),
            pltpu.SemaphoreType.DMA,
        ],
    )
    def gather_kernel(table_hbm, idx_hbm, out_hbm, idx_v, rows_v, sem):
        wid = lax.axis_index("s") * nc + lax.axis_index("c")
        base = wid * per_w
        pltpu.sync_copy(idx_hbm.at[pl.ds(wid * idx_rows, idx_rows)], idx_v)

        def step(g, carry):
            copies = []
            for b in range(FIRE):
                j = g * FIRE + b
                copies.append(
                    pltpu.async_copy(
                        table_hbm.at[idx_v.at[j]],
                        rows_v.at[pl.ds(b * CHUNK, CHUNK)],
                        sem,
                    )
                )
            for c in copies:
                c.wait()
            pltpu.sync_copy(
                rows_v,
                out_hbm.at[pl.ds(base + g * (FIRE * CHUNK), FIRE * CHUNK)],
            )
            return carry

        lax.fori_loop(0, n_steps, step, 0)

    return gather_kernel


def kernel(tok, emb, W1, b1, W2, b2):
    B, L = tok.shape
    n_tokens = B * L
    table = _tanh_layer(emb, W1, b1)
    idx2d = tok.reshape(n_tokens // CHUNK, CHUNK).astype(jnp.int32)
    g = _make_sc_gather(n_tokens)(table, idx2d)
    t = _head(g, W2, b2, B, L)
    return jnp.transpose(t, (2, 0, 1))


# speedup vs baseline: 1.7275x; 1.1112x over previous
import functools

import jax
import jax.numpy as jnp
from jax import lax
from jax.experimental import pallas as pl
from jax.experimental.pallas import tpu as pltpu
from jax.experimental.pallas import tpu_sc as plsc

VOCAB = 100000
D = 128
NQ = 64
ROW_BLK = 2000
OUT_BLK = 4096

NW = 32
CHUNK = 128
FIRE = 4


def _tanh_layer_kernel(emb_ref, w1_ref, b1_ref, h_ref):
    h_ref[...] = jnp.tanh(
        jnp.dot(emb_ref[...], w1_ref[...], preferred_element_type=jnp.float32,
                precision=lax.Precision.DEFAULT)
        + b1_ref[...]
    )


def _tanh_layer(emb, W1, b1):
    grid = VOCAB // ROW_BLK
    return pl.pallas_call(
        _tanh_layer_kernel,
        grid=(grid,),
        in_specs=[
            pl.BlockSpec((ROW_BLK, D), lambda i: (i, 0)),
            pl.BlockSpec((D, D), lambda i: (0, 0)),
            pl.BlockSpec((1, D), lambda i: (0, 0)),
        ],
        out_specs=pl.BlockSpec((ROW_BLK, D), lambda i: (i, 0)),
        out_shape=jax.ShapeDtypeStruct((VOCAB, D), jnp.float32),
    )(emb, W1, b1.reshape(1, D))


def _head_kernel(g_ref, w2_ref, b2_ref, o_ref):
    l_blk = o_ref.shape[0]
    for l in range(l_blk):
        acc = lax.dot_general(
            w2_ref[...], g_ref[:, l, :],
            (((0,), (1,)), ((), ())),
            preferred_element_type=jnp.float32,
            precision=lax.Precision.DEFAULT,
        )
        o_ref[l] = acc + b2_ref[...]


def _head(g, W2, b2, B, L):
    L_BLK = 8
    B_BLK = 1024
    g3 = g.reshape(B, L, D)
    return pl.pallas_call(
        _head_kernel,
        grid=(L // L_BLK, B // B_BLK),
        in_specs=[
            pl.BlockSpec((B_BLK, L_BLK, D), lambda i, j: (j, i, 0)),
            pl.BlockSpec((D, NQ), lambda i, j: (0, 0)),
            pl.BlockSpec((NQ, 1), lambda i, j: (0, 0)),
        ],
        out_specs=pl.BlockSpec((L_BLK, NQ, B_BLK), lambda i, j: (i, 0, j)),
        out_shape=jax.ShapeDtypeStruct((L, NQ, B), jnp.float32),
    )(g3, W2, b2.reshape(NQ, 1))


def _head_chunk_kernel(t_ref, g_ref, w2_ref, b2_ref, o_ref):
    del t_ref
    _head_kernel(g_ref, w2_ref, b2_ref, o_ref)


def _head_chunk(t_in, g, W2, b2, B, L, l0, lc):
    L_BLK = 8
    B_BLK = 1024
    g3 = g.reshape(B, lc, D)
    grid = (lc // L_BLK, B // B_BLK)
    out_spec = pl.BlockSpec(
        (L_BLK, NQ, B_BLK), lambda i, j: (l0 // L_BLK + i, 0, j))
    out_shape = jax.ShapeDtypeStruct((L, NQ, B), jnp.float32)
    g_spec = pl.BlockSpec((B_BLK, L_BLK, D), lambda i, j: (j, i, 0))
    w_spec = pl.BlockSpec((D, NQ), lambda i, j: (0, 0))
    b_spec = pl.BlockSpec((NQ, 1), lambda i, j: (0, 0))
    if t_in is None:
        return pl.pallas_call(
            _head_kernel,
            grid=grid,
            in_specs=[g_spec, w_spec, b_spec],
            out_specs=out_spec,
            out_shape=out_shape,
        )(g3, W2, b2.reshape(NQ, 1))
    return pl.pallas_call(
        _head_chunk_kernel,
        grid=grid,
        in_specs=[
            pl.BlockSpec(memory_space=pl.ANY),
            g_spec, w_spec, b_spec,
        ],
        out_specs=out_spec,
        out_shape=out_shape,
        input_output_aliases={0: 0},
    )(t_in, g3, W2, b2.reshape(NQ, 1))


def _make_sc_gather(n_tokens):
    per_w = n_tokens // NW
    n_steps = per_w // (FIRE * CHUNK)
    idx_rows = per_w // CHUNK

    mesh = plsc.VectorSubcoreMesh(core_axis_name="c", subcore_axis_name="s")
    info = plsc.get_sparse_core_info()
    nc = info.num_cores

    @functools.partial(
        pl.kernel,
        out_type=jax.ShapeDtypeStruct((n_tokens, D), jnp.float32),
        mesh=mesh,
        scratch_types=[
            pltpu.VMEM((idx_rows, CHUNK), jnp.int32),
            pltpu.VMEM((FIRE * CHUNK, D), jnp.float32),
            pltpu.SemaphoreType.DMA,
        ],
    )
    def gather_kernel(table_hbm, idx_hbm, out_hbm, idx_v, rows_v, sem):
        wid = lax.axis_index("s") * nc + lax.axis_index("c")
        base = wid * per_w
        pltpu.sync_copy(idx_hbm.at[pl.ds(wid * idx_rows, idx_rows)], idx_v)

        def step(g, carry):
            copies = []
            for b in range(FIRE):
                j = g * FIRE + b
                copies.append(
                    pltpu.async_copy(
                        table_hbm.at[idx_v.at[j]],
                        rows_v.at[pl.ds(b * CHUNK, CHUNK)],
                        sem,
                    )
                )
            for c in copies:
                c.wait()
            pltpu.sync_copy(
                rows_v,
                out_hbm.at[pl.ds(base + g * (FIRE * CHUNK), FIRE * CHUNK)],
            )
            return carry

        lax.fori_loop(0, n_steps, step, 0)

    return gather_kernel


N_CHUNKS = 5


def kernel(tok, emb, W1, b1, W2, b2):
    B, L = tok.shape
    table = _tanh_layer(emb, W1, b1)
    lc = L // N_CHUNKS
    if L % N_CHUNKS == 0 and lc % 8 == 0 and (B * lc) % (NW * FIRE * CHUNK) == 0:
        sc_gather = _make_sc_gather(B * lc)
        t = None
        for i in range(N_CHUNKS):
            idx = tok[:, i * lc:(i + 1) * lc].reshape(-1, CHUNK)
            idx = idx.astype(jnp.int32)
            g = sc_gather(table, idx)
            t = _head_chunk(t, g, W2, b2, B, L, i * lc, lc)
    else:
        n_tokens = B * L
        idx2d = tok.reshape(n_tokens // CHUNK, CHUNK).astype(jnp.int32)
        g = _make_sc_gather(n_tokens)(table, idx2d)
        t = _head(g, W2, b2, B, L)
    return jnp.transpose(t, (2, 0, 1))


# speedup vs baseline: 1.8513x; 1.0717x over previous
import functools

import jax
import jax.numpy as jnp
from jax import lax
from jax.experimental import pallas as pl
from jax.experimental.pallas import tpu as pltpu
from jax.experimental.pallas import tpu_sc as plsc

VOCAB = 100000
D = 128
NQ = 64
ROW_BLK = 2000
OUT_BLK = 4096

NW = 32
CHUNK = 128
FIRE = 2


def _tanh_layer_kernel(emb_ref, w1_ref, b1_ref, h_ref):
    h_ref[...] = jnp.tanh(
        jnp.dot(emb_ref[...], w1_ref[...], preferred_element_type=jnp.float32,
                precision=lax.Precision.DEFAULT)
        + b1_ref[...]
    )


def _tanh_layer(emb, W1, b1):
    grid = VOCAB // ROW_BLK
    return pl.pallas_call(
        _tanh_layer_kernel,
        grid=(grid,),
        in_specs=[
            pl.BlockSpec((ROW_BLK, D), lambda i: (i, 0)),
            pl.BlockSpec((D, D), lambda i: (0, 0)),
            pl.BlockSpec((1, D), lambda i: (0, 0)),
        ],
        out_specs=pl.BlockSpec((ROW_BLK, D), lambda i: (i, 0)),
        out_shape=jax.ShapeDtypeStruct((VOCAB, D), jnp.float32),
    )(emb, W1, b1.reshape(1, D))


def _head_kernel(g_ref, w2_ref, b2_ref, o_ref):
    l_blk = o_ref.shape[0]
    for l in range(l_blk):
        acc = lax.dot_general(
            w2_ref[...], g_ref[:, l, :],
            (((0,), (1,)), ((), ())),
            preferred_element_type=jnp.float32,
            precision=lax.Precision.DEFAULT,
        )
        o_ref[l] = acc + b2_ref[...]


def _head(g, W2, b2, B, L):
    L_BLK = 8
    B_BLK = 1024
    g3 = g.reshape(B, L, D)
    return pl.pallas_call(
        _head_kernel,
        grid=(L // L_BLK, B // B_BLK),
        in_specs=[
            pl.BlockSpec((B_BLK, L_BLK, D), lambda i, j: (j, i, 0)),
            pl.BlockSpec((D, NQ), lambda i, j: (0, 0)),
            pl.BlockSpec((NQ, 1), lambda i, j: (0, 0)),
        ],
        out_specs=pl.BlockSpec((L_BLK, NQ, B_BLK), lambda i, j: (i, 0, j)),
        out_shape=jax.ShapeDtypeStruct((L, NQ, B), jnp.float32),
    )(g3, W2, b2.reshape(NQ, 1))


def _head_lmajor_kernel(g_ref, w2_ref, b2_ref, o_ref):
    l_blk = o_ref.shape[0]
    for l in range(l_blk):
        acc = lax.dot_general(
            w2_ref[...], g_ref[l],
            (((0,), (1,)), ((), ())),
            preferred_element_type=jnp.float32,
            precision=lax.Precision.DEFAULT,
        )
        o_ref[l] = acc + b2_ref[...]


def _head_chunk_kernel(t_ref, g_ref, w2_ref, b2_ref, o_ref):
    del t_ref
    _head_lmajor_kernel(g_ref, w2_ref, b2_ref, o_ref)


def _head_chunk(t_in, g, W2, b2, B, L, l0, lc):
    L_BLK = 8
    B_BLK = 1024
    g3 = g.reshape(lc, B, D)
    grid = (lc // L_BLK, B // B_BLK)
    out_spec = pl.BlockSpec(
        (L_BLK, NQ, B_BLK), lambda i, j: (l0 // L_BLK + i, 0, j))
    out_shape = jax.ShapeDtypeStruct((L, NQ, B), jnp.float32)
    g_spec = pl.BlockSpec((L_BLK, B_BLK, D), lambda i, j: (i, j, 0))
    w_spec = pl.BlockSpec((D, NQ), lambda i, j: (0, 0))
    b_spec = pl.BlockSpec((NQ, 1), lambda i, j: (0, 0))
    if t_in is None:
        return pl.pallas_call(
            _head_lmajor_kernel,
            grid=grid,
            in_specs=[g_spec, w_spec, b_spec],
            out_specs=out_spec,
            out_shape=out_shape,
        )(g3, W2, b2.reshape(NQ, 1))
    return pl.pallas_call(
        _head_chunk_kernel,
        grid=grid,
        in_specs=[
            pl.BlockSpec(memory_space=pl.ANY),
            g_spec, w_spec, b_spec,
        ],
        out_specs=out_spec,
        out_shape=out_shape,
        input_output_aliases={0: 0},
    )(t_in, g3, W2, b2.reshape(NQ, 1))


def _make_sc_gather(n_tokens):
    per_w = n_tokens // NW
    n_steps = per_w // (FIRE * CHUNK)
    idx_rows = per_w // CHUNK

    mesh = plsc.VectorSubcoreMesh(core_axis_name="c", subcore_axis_name="s")
    info = plsc.get_sparse_core_info()
    nc = info.num_cores

    step_rows = FIRE * CHUNK
    assert n_steps % 2 == 0 and n_steps >= 4

    @functools.partial(
        pl.kernel,
        out_type=jax.ShapeDtypeStruct((n_tokens, D), jnp.float32),
        mesh=mesh,
        scratch_types=[
            pltpu.VMEM((idx_rows, CHUNK), jnp.int32),
            pltpu.VMEM((2, step_rows, D), jnp.float32),
            pltpu.SemaphoreType.DMA,
            pltpu.SemaphoreType.DMA,
            pltpu.SemaphoreType.DMA,
        ],
    )
    def gather_kernel(table_hbm, idx_hbm, out_hbm, idx_v, rows_v, sem_g,
                      sem_w0, sem_w1):
        wid = lax.axis_index("s") * nc + lax.axis_index("c")
        base = wid * per_w
        sem_w = (sem_w0, sem_w1)
        pltpu.sync_copy(idx_hbm.at[pl.ds(wid * idx_rows, idx_rows)], idx_v)

        def fire_and_wait(step, b):
            copies = []
            for f in range(FIRE):
                copies.append(
                    pltpu.async_copy(
                        table_hbm.at[idx_v.at[step * FIRE + f]],
                        rows_v.at[b].at[pl.ds(f * CHUNK, CHUNK)],
                        sem_g,
                    )
                )
            for c in copies:
                c.wait()

        def writeback(step, b):
            pltpu.async_copy(
                rows_v.at[b],
                out_hbm.at[pl.ds(base + step * step_rows, step_rows)],
                sem_w[b],
            )

        def drain(b):
            pltpu.make_async_copy(
                rows_v.at[b],
                out_hbm.at[pl.ds(base, step_rows)],
                sem_w[b],
            ).wait()

        for b in (0, 1):
            fire_and_wait(b, b)
            writeback(b, b)

        def step2(g2, carry):
            for b in (0, 1):
                step = g2 * 2 + b
                drain(b)
                fire_and_wait(step, b)
                writeback(step, b)
            return carry

        lax.fori_loop(1, n_steps // 2, step2, 0)
        drain(0)
        drain(1)

    return gather_kernel


N_CHUNKS = 5


def kernel(tok, emb, W1, b1, W2, b2):
    B, L = tok.shape
    table = _tanh_layer(emb, W1, b1)
    lc = L // N_CHUNKS
    if L % N_CHUNKS == 0 and lc % 8 == 0 and (B * lc) % (NW * FIRE * CHUNK) == 0:
        sc_gather = _make_sc_gather(B * lc)
        t = None
        for i in range(N_CHUNKS):
            idx = tok[:, i * lc:(i + 1) * lc].T.reshape(-1, CHUNK)
            idx = idx.astype(jnp.int32)
            g = sc_gather(table, idx)
            t = _head_chunk(t, g, W2, b2, B, L, i * lc, lc)
    else:
        n_tokens = B * L
        idx2d = tok.reshape(n_tokens // CHUNK, CHUNK).astype(jnp.int32)
        g = _make_sc_gather(n_tokens)(table, idx2d)
        t = _head(g, W2, b2, B, L)
    return jnp.transpose(t, (2, 0, 1))
